# Initial kernel scaffold; baseline (speedup 1.0000x reference)
import jax
import jax.numpy as jnp
from jax.experimental import pallas as pl


def kernel(pos, vel, box, box_feats, Wc0f, Wc0o, Wd0, bd0, Wc1, Wd1, bd1, Wc2, Wd2, bd2, Wc3, Wd3, bd3):
    raise NotImplementedError("write your pallas kernel here")


# speedup vs baseline: 101.5825x; 101.5825x over previous
import functools
import jax
import jax.numpy as jnp
from jax.experimental import pallas as pl
from jax.experimental.pallas import tpu as pltpu

_PARTICLE_RADIUS = 0.025
_RADIUS_SCALE = 1.5
_FILTER_EXTENT = _RADIUS_SCALE * 6.0 * _PARTICLE_RADIUS
_RADIUS = _FILTER_EXTENT / 2.0
_DT = 1.0 / 50.0
_K = 4


def _conv_body(pd_ref, psT_ref, fs_ref, *rest, out_ref, B_ref,
               cin, cout, n_s, exclude_self, relu_feats, has_dense, has_res,
               dense_concat):
    idx = 0
    fd_ref = rest[idx] if has_dense else None
    idx += 1 if has_dense else 0
    ansd_ref = rest[idx] if has_res else None
    idx += 1 if has_res else 0
    wc_ref = rest[idx]
    wd_ref = rest[idx + 1] if has_dense else None
    bd_ref = rest[idx + 2] if has_dense else None

    s = pl.program_id(1)

    dx = psT_ref[0:1, :] - pd_ref[:, 0:1]
    dy = psT_ref[1:2, :] - pd_ref[:, 1:2]
    dz = psT_ref[2:3, :] - pd_ref[:, 2:3]
    d2 = dx * dx + dy * dy + dz * dz
    r2max = _RADIUS * _RADIUS
    m = d2 < r2max
    if exclude_self:
        m = jnp.logical_and(m, d2 > 1e-12)
    maskf = m.astype(jnp.float32)

    inv_r2 = 1.0 / (_RADIUS * _RADIUS)
    r2 = d2 * inv_r2
    win = jnp.clip((1.0 - r2) ** 3, 0.0, 1.0) * maskf

    inv_r = 1.0 / _RADIUS
    scale = 0.5 * (_K - 1)
    tx = (jnp.clip(dx * inv_r, -1.0, 1.0) + 1.0) * scale
    ty = (jnp.clip(dy * inv_r, -1.0, 1.0) + 1.0) * scale
    tz = (jnp.clip(dz * inv_r, -1.0, 1.0) + 1.0) * scale

    hx = [jnp.maximum(1.0 - jnp.abs(tx - a), 0.0) for a in range(_K)]
    hy = [jnp.maximum(1.0 - jnp.abs(ty - a), 0.0) for a in range(_K)]
    hz = [jnp.maximum(1.0 - jnp.abs(tz - a), 0.0) * win for a in range(_K)]

    F = fs_ref[...]
    if relu_feats:
        F = jnp.maximum(F, 0.0)

    @pl.when(s == 0)
    def _():
        B_ref[...] = jnp.zeros_like(B_ref)

    for a in range(_K):
        for b in range(_K):
            hxy = hx[a] * hy[b]
            for g in range(_K):
                coef = hxy * hz[g]
                k = a * 16 + b * 4 + g
                B_ref[:, k * cin:(k + 1) * cin] += jnp.dot(
                    coef, F, preferred_element_type=jnp.float32)

    @pl.when(s == n_s - 1)
    def _():
        acc = jnp.dot(B_ref[...], wc_ref[...],
                      preferred_element_type=jnp.float32)
        if has_dense:
            fd = fd_ref[...]
            if relu_feats:
                fd = jnp.maximum(fd, 0.0)
            ad = jnp.dot(fd, wd_ref[...],
                         preferred_element_type=jnp.float32) + bd_ref[...]
            if dense_concat:
                acc = jnp.concatenate([acc, ad], axis=-1)
            else:
                acc = acc + ad
        if has_res:
            acc = acc + ansd_ref[...]
        out_ref[...] = acc


def _cconv(pos_dst, pos_srcT, feats_src, Wc, feats_dst=None, Wd=None, bd=None,
           ans_dst=None, exclude_self=True, relu_feats=False,
           dense_concat=False, nt=256, S=512):
    Nd = pos_dst.shape[0]
    Ns = pos_srcT.shape[1]
    cin = feats_src.shape[1]
    cout = Wc.shape[-1]
    dense_w = Wd.shape[1] if Wd is not None else 0
    out_w = cout + (dense_w if dense_concat else 0)
    n_d = Nd // nt
    n_s = Ns // S
    has_dense = Wd is not None
    has_res = ans_dst is not None

    Wc2 = Wc.reshape(_K * _K * _K * cin, cout)

    in_specs = [
        pl.BlockSpec((nt, 3), lambda d, s: (d, 0)),
        pl.BlockSpec((3, S), lambda d, s: (0, s)),
        pl.BlockSpec((S, cin), lambda d, s: (s, 0)),
    ]
    args = [pos_dst, pos_srcT, feats_src]
    if has_dense:
        cd = feats_dst.shape[1]
        in_specs.append(pl.BlockSpec((nt, cd), lambda d, s: (d, 0)))
        args.append(feats_dst)
    if has_res:
        in_specs.append(pl.BlockSpec((nt, cout), lambda d, s: (d, 0)))
        args.append(ans_dst)
    in_specs.append(pl.BlockSpec(Wc2.shape, lambda d, s: (0, 0)))
    args.append(Wc2)
    if has_dense:
        in_specs.append(pl.BlockSpec(Wd.shape, lambda d, s: (0, 0)))
        args.append(Wd)
        in_specs.append(pl.BlockSpec((1, dense_w), lambda d, s: (0, 0)))
        args.append(bd.reshape(1, dense_w))

    body = functools.partial(
        _conv_body, cin=cin, cout=cout, n_s=n_s, exclude_self=exclude_self,
        relu_feats=relu_feats, has_dense=has_dense, has_res=has_res,
        dense_concat=dense_concat)

    def wrapped(*refs):
        n_in = len(args)
        body(*refs[:n_in], out_ref=refs[n_in], B_ref=refs[n_in + 1])

    return pl.pallas_call(
        wrapped,
        grid=(n_d, n_s),
        in_specs=in_specs,
        out_specs=pl.BlockSpec((nt, out_w), lambda d, s: (d, 0)),
        out_shape=jax.ShapeDtypeStruct((Nd, out_w), jnp.float32),
        scratch_shapes=[pltpu.VMEM((nt, _K * _K * _K * cin), jnp.float32)],
        compiler_params=pltpu.CompilerParams(
            dimension_semantics=("parallel", "arbitrary")),
    )(*args)


def _pad_rows(x, n, val):
    pad = n - x.shape[0]
    if pad == 0:
        return x
    return jnp.concatenate(
        [x, jnp.full((pad,) + x.shape[1:], val, x.dtype)], axis=0)


def kernel(pos, vel, box, box_feats, Wc0f, Wc0o, Wd0, bd0, Wc1, Wd1, bd1,
           Wc2, Wd2, bd2, Wc3, Wd3, bd3):
    n = pos.shape[0]
    gravity = jnp.array([0.0, -9.81, 0.0], dtype=jnp.float32)
    vel2 = vel + _DT * gravity
    pos2 = pos + _DT * (vel2 + vel) / 2.0
    fluid_feats = jnp.concatenate([jnp.ones_like(pos2[:, 0:1]), vel2], axis=-1)

    NT, SB = 256, 512
    Nd = ((n + NT - 1) // NT) * NT
    Ns = ((n + SB - 1) // SB) * SB
    Nd = max(Nd, Ns)
    Ns = Nd
    nb = box.shape[0]
    Nb = ((nb + SB - 1) // SB) * SB

    pos_p = _pad_rows(pos2, Nd, 1e9)
    posT = pos_p.T
    ff_p = _pad_rows(fluid_feats, Nd, 0.0)
    box_p = _pad_rows(box, Nb, 1e9)
    boxT = box_p.T
    bf_p = _pad_rows(box_feats, Nb, 0.0)

    a_obst = _cconv(pos_p, boxT, bf_p, Wc0o, exclude_self=False,
                    nt=NT, S=SB)
    a_c0 = _cconv(pos_p, posT, ff_p, Wc0f, feats_dst=ff_p, Wd=Wd0, bd=bd0,
                  exclude_self=True, dense_concat=True, nt=NT, S=SB)
    ans = jnp.concatenate([a_obst, a_c0], axis=-1)

    for Wc, Wd, bd, res in [(Wc1, Wd1, bd1, False), (Wc2, Wd2, bd2, True),
                            (Wc3, Wd3, bd3, False)]:
        ans = _cconv(pos_p, posT, ans, Wc, feats_dst=ans, Wd=Wd, bd=bd,
                     ans_dst=ans if res else None,
                     exclude_self=True, relu_feats=True, nt=NT, S=SB)

    return (1.0 / 128.0) * ans[:n]
